# baseline (device time: 18019 ns/iter reference)
import jax
import jax.numpy as jnp
from jax import lax
from jax.experimental import pallas as pl
from jax.experimental.pallas import tpu as pltpu

N_DEV = 4
N_TOK = 512
D_IN = 256
D_OUT = 512
N_EXP = 8
CAP = 51
SLOTS = 64
E_LOCAL = 2
BLK = E_LOCAL * SLOTS


def kernel(x, router_W, route_idx, expert_W):
    del router_W

    def body(x_ref, idx_ref, w_ref, out_ref, gather_ref, comm_ref,
             send_sems, recv_sems):
        my = lax.axis_index("i")
        left = lax.rem(my + N_DEV - 1, N_DEV)
        right = lax.rem(my + 1, N_DEV)

        barrier_sem = pltpu.get_barrier_semaphore()
        for nbr in [left, right]:
            pl.semaphore_signal(
                barrier_sem, inc=1,
                device_id=(nbr,), device_id_type=pl.DeviceIdType.MESH,
            )
        pl.semaphore_wait(barrier_sem, 2)

        idx = idx_ref[...]
        e_iota = lax.broadcasted_iota(jnp.int32, (N_TOK, N_EXP), 1)
        onehot = (idx == e_iota).astype(jnp.float32)

        ti = lax.broadcasted_iota(jnp.int32, (N_TOK, N_TOK), 0)
        tj = lax.broadcasted_iota(jnp.int32, (N_TOK, N_TOK), 1)
        ltri = (tj < ti).astype(jnp.float32)
        cum_before = lax.dot_general(
            ltri, onehot, (((1,), (0,)), ((), ())),
            preferred_element_type=jnp.float32,
        )
        rank_f = jnp.sum(cum_before * onehot, axis=1, keepdims=True)
        rank = rank_f.astype(jnp.int32)
        keep = rank < CAP

        s_iota = lax.broadcasted_iota(jnp.int32, (N_TOK, BLK), 1)
        s_exp = my * E_LOCAL + s_iota // SLOTS
        s_rank = lax.rem(s_iota, SLOTS)
        gt = jnp.where(
            keep & (idx == s_exp) & (rank == s_rank), 1.0, 0.0
        ).astype(jnp.bfloat16)

        xb = x_ref[...].astype(jnp.bfloat16)
        xg = lax.dot_general(
            gt, xb, (((0,), (0,)), ((), ())),
            preferred_element_type=jnp.float32,
        ).astype(jnp.bfloat16)

        c0 = lax.dot_general(
            xg[0:SLOTS, :], w_ref[0].astype(jnp.bfloat16),
            (((1,), (0,)), ((), ())), preferred_element_type=jnp.float32,
        ).astype(jnp.bfloat16)
        c1 = lax.dot_general(
            xg[SLOTS:BLK, :], w_ref[1].astype(jnp.bfloat16),
            (((1,), (0,)), ((), ())), preferred_element_type=jnp.float32,
        ).astype(jnp.bfloat16)

        comm_ref[0, 0:SLOTS, :] = c0
        comm_ref[0, SLOTS:BLK, :] = c1
        gather_ref[pl.ds(my * BLK, BLK), :] = comm_ref[0]

        for h in range(N_DEV - 1):
            send_slot = h % 2
            recv_slot = (h + 1) % 2
            rdma = pltpu.make_async_remote_copy(
                src_ref=comm_ref.at[send_slot],
                dst_ref=comm_ref.at[recv_slot],
                send_sem=send_sems.at[send_slot],
                recv_sem=recv_sems.at[recv_slot],
                device_id=(right,),
                device_id_type=pl.DeviceIdType.MESH,
            )
            rdma.start()
            rdma.wait()

            origin = lax.rem(my + N_DEV - 1 - h, N_DEV)
            gather_ref[pl.ds(origin * BLK, BLK), :] = comm_ref[recv_slot]

        slot = idx * SLOTS + rank
        g_iota = lax.broadcasted_iota(jnp.int32, (N_TOK, N_EXP * SLOTS), 1)
        scat = jnp.where(keep & (g_iota == slot), 1.0, 0.0).astype(jnp.bfloat16)
        out_ref[...] = lax.dot_general(
            scat, gather_ref[...], (((1,), (0,)), ((), ())),
            preferred_element_type=jnp.float32,
        )

    return pl.pallas_call(
        body,
        out_shape=jax.ShapeDtypeStruct((N_TOK, D_OUT), jnp.float32),
        in_specs=[
            pl.BlockSpec(memory_space=pltpu.VMEM),
            pl.BlockSpec(memory_space=pltpu.VMEM),
            pl.BlockSpec(memory_space=pltpu.VMEM),
        ],
        out_specs=pl.BlockSpec(memory_space=pltpu.VMEM),
        scratch_shapes=[
            pltpu.VMEM((N_EXP * SLOTS, D_OUT), jnp.bfloat16),
            pltpu.VMEM((2, BLK, D_OUT), jnp.bfloat16),
            pltpu.SemaphoreType.DMA((2,)),
            pltpu.SemaphoreType.DMA((2,)),
        ],
        compiler_params=pltpu.CompilerParams(collective_id=0),
    )(x, route_idx, expert_W)


# device time: 12520 ns/iter; 1.4392x vs baseline; 1.4392x over previous
import jax
import jax.numpy as jnp
from jax import lax
from jax.experimental import pallas as pl
from jax.experimental.pallas import tpu as pltpu

N_DEV = 4
N_TOK = 512
D_IN = 256
D_OUT = 512
N_EXP = 8
CAP = 51
SLOTS = 64
E_LOCAL = 2
BLK = E_LOCAL * SLOTS


def kernel(x, router_W, route_idx, expert_W):
    del router_W

    def body(x_ref, idx_ref, w_ref, out_ref, gather_ref, send_sems, recv_sems):
        my = lax.axis_index("i")
        peers = [lax.rem(my + d, N_DEV) for d in (1, 2, 3)]

        barrier_sem = pltpu.get_barrier_semaphore()
        for nbr in peers:
            pl.semaphore_signal(
                barrier_sem, inc=1,
                device_id=(nbr,), device_id_type=pl.DeviceIdType.MESH,
            )

        idx = idx_ref[...]
        e_iota = lax.broadcasted_iota(jnp.int32, (N_TOK, N_EXP), 1)
        onehot = (idx == e_iota).astype(jnp.float32)

        ti = lax.broadcasted_iota(jnp.int32, (N_TOK, N_TOK), 0)
        tj = lax.broadcasted_iota(jnp.int32, (N_TOK, N_TOK), 1)
        ltri = (tj < ti).astype(jnp.float32)
        cum_before = lax.dot_general(
            ltri, onehot, (((1,), (0,)), ((), ())),
            preferred_element_type=jnp.float32,
        )
        rank_f = jnp.sum(cum_before * onehot, axis=1, keepdims=True)
        rank = rank_f.astype(jnp.int32)
        keep = rank < CAP

        s_iota = lax.broadcasted_iota(jnp.int32, (N_TOK, BLK), 1)
        s_exp = my * E_LOCAL + s_iota // SLOTS
        s_rank = lax.rem(s_iota, SLOTS)
        gt = jnp.where(
            keep & (idx == s_exp) & (rank == s_rank), 1.0, 0.0
        ).astype(jnp.bfloat16)

        xb = x_ref[...].astype(jnp.bfloat16)
        xg = lax.dot_general(
            gt, xb, (((0,), (0,)), ((), ())),
            preferred_element_type=jnp.float32,
        ).astype(jnp.bfloat16)

        gather_ref[my, 0:SLOTS, :] = lax.dot_general(
            xg[0:SLOTS, :], w_ref[0].astype(jnp.bfloat16),
            (((1,), (0,)), ((), ())), preferred_element_type=jnp.float32,
        ).astype(jnp.bfloat16)
        gather_ref[my, SLOTS:BLK, :] = lax.dot_general(
            xg[SLOTS:BLK, :], w_ref[1].astype(jnp.bfloat16),
            (((1,), (0,)), ((), ())), preferred_element_type=jnp.float32,
        ).astype(jnp.bfloat16)

        pl.semaphore_wait(barrier_sem, N_DEV - 1)

        sends = []
        for j, p in enumerate(peers):
            rdma = pltpu.make_async_remote_copy(
                src_ref=gather_ref.at[my],
                dst_ref=gather_ref.at[my],
                send_sem=send_sems.at[j],
                recv_sem=recv_sems.at[my],
                device_id=(p,),
                device_id_type=pl.DeviceIdType.MESH,
            )
            rdma.start()
            sends.append(rdma)

        slot = idx * SLOTS + rank
        g_iota = lax.broadcasted_iota(jnp.int32, (N_TOK, N_EXP * SLOTS), 1)
        scat = jnp.where(keep & (g_iota == slot), 1.0, 0.0).astype(jnp.bfloat16)

        for p in peers:
            recv = pltpu.make_async_remote_copy(
                src_ref=gather_ref.at[my],
                dst_ref=gather_ref.at[p],
                send_sem=send_sems.at[0],
                recv_sem=recv_sems.at[p],
                device_id=(p,),
                device_id_type=pl.DeviceIdType.MESH,
            )
            recv.wait_recv()

        gathered = gather_ref[...].reshape(N_EXP * SLOTS, D_OUT)
        out_ref[...] = lax.dot_general(
            scat, gathered, (((1,), (0,)), ((), ())),
            preferred_element_type=jnp.float32,
        )

        for rdma in sends:
            rdma.wait_send()

    return pl.pallas_call(
        body,
        out_shape=jax.ShapeDtypeStruct((N_TOK, D_OUT), jnp.float32),
        in_specs=[
            pl.BlockSpec(memory_space=pltpu.VMEM),
            pl.BlockSpec(memory_space=pltpu.VMEM),
            pl.BlockSpec(memory_space=pltpu.VMEM),
        ],
        out_specs=pl.BlockSpec(memory_space=pltpu.VMEM),
        scratch_shapes=[
            pltpu.VMEM((N_DEV, BLK, D_OUT), jnp.bfloat16),
            pltpu.SemaphoreType.DMA((N_DEV - 1,)),
            pltpu.SemaphoreType.DMA((N_DEV,)),
        ],
        compiler_params=pltpu.CompilerParams(collective_id=0),
    )(x, route_idx, expert_W)


# device time: 10214 ns/iter; 1.7641x vs baseline; 1.2258x over previous
import jax
import jax.numpy as jnp
from jax import lax
from jax.experimental import pallas as pl
from jax.experimental.pallas import tpu as pltpu

N_DEV = 4
N_TOK = 512
D_IN = 256
D_OUT = 512
N_EXP = 8
CAP = 51
E_LOCAL = 2
ROWS = 112
G_ROWS = N_DEV * ROWS


def kernel(x, router_W, route_idx, expert_W):
    del router_W
    idx_row = route_idx.reshape(1, N_TOK)
    xb16 = x.astype(jnp.bfloat16)
    wb16 = expert_W.astype(jnp.bfloat16)

    def body(x_ref, idx_ref, idxr_ref, w_ref, out_ref, gather_ref,
             send_sems, recv_sems):
        my = lax.axis_index("i")
        peers = [lax.rem(my + d, N_DEV) for d in (1, 2, 3)]

        barrier_sem = pltpu.get_barrier_semaphore()
        for nbr in peers:
            pl.semaphore_signal(
                barrier_sem, inc=1,
                device_id=(nbr,), device_id_type=pl.DeviceIdType.MESH,
            )

        ti = lax.broadcasted_iota(jnp.int32, (N_TOK, N_TOK), 0)
        tj = lax.broadcasted_iota(jnp.int32, (N_TOK, N_TOK), 1)
        utri = (tj > ti).astype(jnp.bfloat16)

        idx_r = idxr_ref[...]
        e_iota_r = lax.broadcasted_iota(jnp.int32, (N_EXP, N_TOK), 0)
        onehot_r = (idx_r == e_iota_r).astype(jnp.bfloat16)
        cum_r = lax.dot_general(
            onehot_r, utri, (((1,), (0,)), ((), ())),
            preferred_element_type=jnp.float32,
        )
        rank_r = jnp.sum(cum_r * onehot_r.astype(jnp.float32), axis=0,
                         keepdims=True)
        keep_r = rank_r < float(CAP)

        s_iota = lax.broadcasted_iota(jnp.int32, (ROWS, N_TOK), 0)
        is_e1 = (s_iota >= CAP).astype(jnp.int32)
        s_exp = my * E_LOCAL + is_e1
        s_rank = (s_iota - CAP * is_e1).astype(jnp.float32)
        valid = s_iota < E_LOCAL * CAP
        g_oh = jnp.where(
            valid & keep_r & (idx_r == s_exp) & (rank_r == s_rank), 1.0, 0.0
        ).astype(jnp.bfloat16)

        xg = lax.dot_general(
            g_oh, x_ref[...], (((1,), (0,)), ((), ())),
            preferred_element_type=jnp.float32,
        ).astype(jnp.bfloat16)

        y0 = lax.dot_general(
            xg, w_ref[0], (((1,), (0,)), ((), ())),
            preferred_element_type=jnp.float32,
        )
        y1 = lax.dot_general(
            xg, w_ref[1], (((1,), (0,)), ((), ())),
            preferred_element_type=jnp.float32,
        )
        r_io = lax.broadcasted_iota(jnp.int32, (ROWS, D_OUT), 0)
        gather_ref[my, :, :] = jnp.where(
            r_io < CAP, y0, jnp.where(r_io < E_LOCAL * CAP, y1, 0.0)
        ).astype(jnp.bfloat16)

        pl.semaphore_wait(barrier_sem, N_DEV - 1)

        sends = []
        for j, p in enumerate(peers):
            rdma = pltpu.make_async_remote_copy(
                src_ref=gather_ref.at[my],
                dst_ref=gather_ref.at[my],
                send_sem=send_sems.at[j],
                recv_sem=recv_sems.at[my],
                device_id=(p,),
                device_id_type=pl.DeviceIdType.MESH,
            )
            rdma.start()
            sends.append(rdma)

        idx_c = idx_ref[...]
        e_iota_c = lax.broadcasted_iota(jnp.int32, (N_TOK, N_EXP), 1)
        onehot_c = (idx_c == e_iota_c).astype(jnp.bfloat16)
        ltri = (tj < ti).astype(jnp.bfloat16)
        cum_c = lax.dot_general(
            ltri, onehot_c, (((1,), (0,)), ((), ())),
            preferred_element_type=jnp.float32,
        )
        rank_c = jnp.sum(cum_c * onehot_c.astype(jnp.float32), axis=1,
                         keepdims=True).astype(jnp.int32)
        keep_c = rank_c < CAP
        slot = (idx_c // E_LOCAL) * ROWS + lax.rem(idx_c, E_LOCAL) * CAP + rank_c
        g2 = lax.broadcasted_iota(jnp.int32, (N_TOK, G_ROWS), 1)
        scat = jnp.where(keep_c & (g2 == slot), 1.0, 0.0).astype(jnp.bfloat16)

        for p in peers:
            recv = pltpu.make_async_remote_copy(
                src_ref=gather_ref.at[my],
                dst_ref=gather_ref.at[p],
                send_sem=send_sems.at[0],
                recv_sem=recv_sems.at[p],
                device_id=(p,),
                device_id_type=pl.DeviceIdType.MESH,
            )
            recv.wait_recv()

        gathered = gather_ref[...].reshape(G_ROWS, D_OUT)
        out_ref[...] = lax.dot_general(
            scat, gathered, (((1,), (0,)), ((), ())),
            preferred_element_type=jnp.float32,
        ).astype(jnp.bfloat16)

        for rdma in sends:
            rdma.wait_send()

    return pl.pallas_call(
        body,
        out_shape=jax.ShapeDtypeStruct((N_TOK, D_OUT), jnp.bfloat16),
        in_specs=[
            pl.BlockSpec(memory_space=pltpu.VMEM),
            pl.BlockSpec(memory_space=pltpu.VMEM),
            pl.BlockSpec(memory_space=pltpu.VMEM),
            pl.BlockSpec(memory_space=pltpu.VMEM),
        ],
        out_specs=pl.BlockSpec(memory_space=pltpu.VMEM),
        scratch_shapes=[
            pltpu.VMEM((N_DEV, ROWS, D_OUT), jnp.bfloat16),
            pltpu.SemaphoreType.DMA((N_DEV - 1,)),
            pltpu.SemaphoreType.DMA((N_DEV,)),
        ],
        compiler_params=pltpu.CompilerParams(collective_id=0),
    )(xb16, route_idx, idx_row, wb16)
